# baseline (speedup 1.0000x reference)
import functools

import jax
import jax.numpy as jnp
from jax import lax
from jax.experimental import pallas as pl
from jax.experimental.pallas import tpu as pltpu
from jax.experimental.pallas import tpu_sc as plsc

D_MODEL = 128
SCALE = float(D_MODEL) ** 0.5

NC = 2
NS = 16
NW = NC * NS
G = 128
NB = 2


@functools.lru_cache(maxsize=None)
def _emb_kernel(B: int):
    n_per_w = B // NW
    n_groups = n_per_w // G
    assert n_groups % NB == 0

    mesh = plsc.VectorSubcoreMesh(core_axis_name="c", subcore_axis_name="s")

    @functools.partial(
        pl.kernel,
        mesh=mesh,
        out_type=jax.ShapeDtypeStruct((B, D_MODEL), jnp.float32),
        scratch_types=[
            pltpu.VMEM((n_groups, G), jnp.int32),
            *[pltpu.VMEM((G, D_MODEL), jnp.float32) for _ in range(2 * NB)],
            *[pltpu.SemaphoreType.DMA for _ in range(2 * NB)],
        ],
    )
    def k(x_hbm, table_hbm, out_hbm, idx_v, *bufs_and_sems):
        inb = bufs_and_sems[0:NB]
        outb = bufs_and_sems[NB:2 * NB]
        gsem = bufs_and_sems[2 * NB:3 * NB]
        osem = bufs_and_sems[3 * NB:4 * NB]

        wid = lax.axis_index("s") * NC + lax.axis_index("c")
        base = wid * n_per_w
        pltpu.sync_copy(x_hbm.at[wid], idx_v)

        for b in range(NB):
            pltpu.async_copy(table_hbm.at[idx_v.at[b]], inb[b], gsem[b])

        def step(i, carry):
            for b in range(NB):
                gg = i * NB + b
                pltpu.make_async_copy(table_hbm.at[idx_v.at[gg]],
                                      inb[b], gsem[b]).wait()
                @pl.when(gg >= NB)
                def _():
                    pltpu.make_async_copy(
                        outb[b],
                        out_hbm.at[pl.ds(base + (gg - NB) * G, G)],
                        osem[b]).wait()

                def row(r, c):
                    for j in range(D_MODEL // 16):
                        sl = pl.ds(j * 16, 16)
                        outb[b][r, sl] = inb[b][r, sl] * SCALE
                    return c

                lax.fori_loop(0, G, row, 0, unroll=2)

                @pl.when(gg + NB < n_groups)
                def _():
                    pltpu.async_copy(table_hbm.at[idx_v.at[gg + NB]],
                                     inb[b], gsem[b])

                pltpu.async_copy(outb[b],
                                 out_hbm.at[pl.ds(base + gg * G, G)],
                                 osem[b])
            return carry

        lax.fori_loop(0, n_groups // NB, step, 0)

        for b in range(NB):
            gg = n_groups - NB + b
            pltpu.make_async_copy(outb[b],
                                  out_hbm.at[pl.ds(base + gg * G, G)],
                                  osem[b]).wait()

    return k


def kernel(x, table):
    s0, s1 = x.shape
    B = s0 * s1
    xi = x.reshape(NW, B // (NW * G), G).astype(jnp.int32)
    out = _emb_kernel(B)(xi, table)
    return out.reshape(s0, s1, D_MODEL)


# speedup vs baseline: 1.8136x; 1.8136x over previous
import functools

import jax
import jax.numpy as jnp
from jax import lax
from jax.experimental import pallas as pl
from jax.experimental.pallas import tpu as pltpu
from jax.experimental.pallas import tpu_sc as plsc

D_MODEL = 128
SCALE = float(D_MODEL) ** 0.5

NC = 2
NS = 16
NW = NC * NS
G = 128
NB = 2


@functools.lru_cache(maxsize=None)
def _emb_kernel(B: int):
    n_per_w = B // NW
    n_groups = n_per_w // G
    assert n_groups % NB == 0

    mesh = plsc.VectorSubcoreMesh(core_axis_name="c", subcore_axis_name="s")

    @functools.partial(
        pl.kernel,
        mesh=mesh,
        out_type=jax.ShapeDtypeStruct((B, D_MODEL), jnp.float32),
        scratch_types=[
            pltpu.VMEM((n_groups, G), jnp.int32),
            *[pltpu.VMEM((G, D_MODEL), jnp.float32) for _ in range(NB)],
            *[pltpu.SemaphoreType.DMA for _ in range(NB)],
        ],
    )
    def k(x_hbm, table_hbm, out_hbm, idx_v, *bufs_and_sems):
        rows = bufs_and_sems[0:NB]
        osem = bufs_and_sems[NB:2 * NB]

        wid = lax.axis_index("s") * NC + lax.axis_index("c")
        base = wid * n_per_w
        pltpu.sync_copy(x_hbm.at[wid], idx_v)

        def step(i, carry):
            for b in range(NB):
                gg = i * NB + b

                @pl.when(gg >= NB)
                def _():
                    pltpu.make_async_copy(
                        rows[b],
                        out_hbm.at[pl.ds(base + (gg - NB) * G, G)],
                        osem[b]).wait()

                pltpu.async_copy(table_hbm.at[idx_v.at[gg]], rows[b],
                                 osem[b]).wait()

                def row(r, c):
                    for j in range(D_MODEL // 16):
                        sl = pl.ds(j * 16, 16)
                        rows[b][r, sl] = rows[b][r, sl] * SCALE
                    return c

                lax.fori_loop(0, G, row, 0)

                pltpu.async_copy(rows[b],
                                 out_hbm.at[pl.ds(base + gg * G, G)],
                                 osem[b])
            return carry

        lax.fori_loop(0, n_groups // NB, step, 0)

        for b in range(NB):
            gg = n_groups - NB + b
            pltpu.make_async_copy(rows[b],
                                  out_hbm.at[pl.ds(base + gg * G, G)],
                                  osem[b]).wait()

    return k


def kernel(x, table):
    s0, s1 = x.shape
    B = s0 * s1
    xi = x.reshape(NW, B // (NW * G), G).astype(jnp.int32)
    out = _emb_kernel(B)(xi, table)
    return out.reshape(s0, s1, D_MODEL)
